# baseline (speedup 1.0000x reference)
import functools

import jax
import jax.numpy as jnp
from jax.experimental import pallas as pl
from jax.experimental.pallas import tpu as pltpu
from jax.experimental.pallas import tpu_sc as plsc

_NSAMPLE_USED = 10
_GATHER_W = 128


def _knn_body(posT_ref, posP_ref, idx_ref, *, base):
    a = posT_ref[0]
    pm = posP_ref[0]
    n = pm.shape[1]
    sq_col = jnp.sum(pm * pm, axis=0, keepdims=True)
    sq_row = jnp.sum(a * a, axis=1, keepdims=True)
    d = jax.lax.dot_general(
        a.astype(jnp.bfloat16), pm.astype(jnp.bfloat16),
        (((1,), (0,)), ((), ())),
        preferred_element_type=jnp.float32)
    w = (sq_row + sq_col) - 2.0 * d
    iota = jax.lax.broadcasted_iota(jnp.int32, w.shape, 1)
    for j in range(_NSAMPLE_USED):
        amin = jnp.argmin(w, axis=1).astype(jnp.int32)
        idx_ref[0, :, j] = amin + base
        w = jnp.where(iota == amin[:, None], jnp.float32(jnp.inf), w)


def _knn_pallas(pos, batch_offset=0, tile_rows=256):
    B, C, N = pos.shape
    posP = jnp.pad(pos, ((0, 0), (0, 8 - C), (0, 0)))
    posT8 = posP.transpose(0, 2, 1)
    R = tile_rows
    return pl.pallas_call(
        functools.partial(_knn_body, base=batch_offset),
        grid=(B, N // R),
        in_specs=[
            pl.BlockSpec((1, R, 8), lambda b, t: (b, t, 0)),
            pl.BlockSpec((1, 8, N), lambda b, t: (b, 0, 0)),
        ],
        out_specs=pl.BlockSpec((1, R, _NSAMPLE_USED), lambda b, t: (b, t, 0)),
        out_shape=jax.ShapeDtypeStruct((B, N, _NSAMPLE_USED), jnp.int32),
        compiler_params=pltpu.CompilerParams(
            dimension_semantics=("parallel", "parallel")),
    )(posT8, posP)


def _sc_gather(table, flat_idx):
    num = flat_idx.shape[1]
    window = 256
    mesh = plsc.VectorSubcoreMesh(core_axis_name="core",
                                  subcore_axis_name="subcore")

    @pl.kernel(out_type=jax.ShapeDtypeStruct((num, _GATHER_W), jnp.float32),
               mesh=mesh)
    def gk(x_hbm, i_hbm, o_hbm):
        def body(i_vmem, o_vmem):
            pltpu.sync_copy(x_hbm.at[i_vmem.at[0]], o_vmem)

        pltpu.emit_pipeline(
            body,
            grid=(num // window,),
            in_specs=[pl.BlockSpec((1, window), lambda i: (0, i))],
            out_specs=[pl.BlockSpec((window, _GATHER_W), lambda i: (i, 0))],
            core_axis_name=("core", "subcore"),
            dimension_semantics=(pltpu.PARALLEL,),
        )(i_hbm, o_hbm)

    return gk(table, flat_idx)


def _fold_weights(W1, b1, W2, b2):
    inds_pos = jnp.array([0, 1, 2, 3, 2, 3, 4, 5, 4, 5, 6, 7, 6, 7, 8, 9],
                         dtype=jnp.int32)
    W2m = W2[:, :, 0, 0]

    def lin(gvec):
        g16 = gvec[:, inds_pos].reshape(1, 3, 2, 2, 2, 2)
        tl = g16[:, :, :, 0, :, 0]
        br = g16[:, :, :, 1, :, 1]
        tr = g16[:, :, :, 0, :, 1]
        bl = g16[:, :, :, 1, :, 0]
        xs = jnp.concatenate([tl, br, tr, bl], axis=1)
        h = jnp.einsum('bchw,ochw->bo', xs, W1)[0]
        return W2m @ h

    basis = jnp.eye(30, dtype=jnp.float32).reshape(30, 3, 10)
    cols = jax.vmap(lin)(basis)
    cvec = W2m @ b1 + b2
    rows = jnp.array([(e % 10) * _GATHER_W + e // 10 for e in range(30)],
                     dtype=jnp.int32)
    A = jnp.zeros((_NSAMPLE_USED * _GATHER_W, 64), jnp.float32).at[rows].set(cols)
    return A.T, cvec


def _mlp_body(q_ref, At_ref, c_ref, o_ref):
    acc = c_ref[...]
    r = None
    for j in range(_NSAMPLE_USED):
        pj = jax.lax.dot_general(
            At_ref[:, j * _GATHER_W:(j + 1) * _GATHER_W], q_ref[j],
            (((1,), (1,)), ((), ())),
            preferred_element_type=jnp.float32,
            precision=jax.lax.Precision.HIGHEST)
        r = pj if r is None else r + pj
    o_ref[0] = r + acc


def _mlp_pallas(q3, At, cvec, B, N):
    R2 = 1024
    nt = N // R2
    return pl.pallas_call(
        _mlp_body,
        grid=(B, nt),
        in_specs=[
            pl.BlockSpec((_NSAMPLE_USED, R2, _GATHER_W),
                         lambda b, t, nt=nt: (0, b * nt + t, 0)),
            pl.BlockSpec((64, _NSAMPLE_USED * _GATHER_W), lambda b, t: (0, 0)),
            pl.BlockSpec((64, 1), lambda b, t: (0, 0)),
        ],
        out_specs=pl.BlockSpec((1, 64, R2), lambda b, t: (b, 0, t)),
        out_shape=jax.ShapeDtypeStruct((B, 64, N), jnp.float32),
        compiler_params=pltpu.CompilerParams(
            dimension_semantics=("parallel", "parallel")),
    )(q3, At, cvec[:, None])


def kernel(pos, W1, b1, W2, b2):
    B, C, N = pos.shape
    table = jnp.pad(pos.transpose(0, 2, 1).reshape(B * N, C),
                    ((0, 0), (0, _GATHER_W - C)))
    At, cvec = _fold_weights(W1, b1, W2, b2)
    outs = []
    for b in range(B):
        gidx = _knn_pallas(pos[b:b + 1], batch_offset=b * N)
        flat_idx = gidx.reshape(N, _NSAMPLE_USED).T.reshape(1, -1)
        qg = _sc_gather(table, flat_idx)
        q3 = qg.reshape(_NSAMPLE_USED, N, _GATHER_W)
        outs.append(_mlp_pallas(q3, At, cvec, 1, N))
    return jnp.concatenate(outs, axis=0)


# speedup vs baseline: 1.0055x; 1.0055x over previous
import functools

import jax
import jax.numpy as jnp
from jax.experimental import pallas as pl
from jax.experimental.pallas import tpu as pltpu
from jax.experimental.pallas import tpu_sc as plsc

_NSAMPLE_USED = 10
_GATHER_W = 128


def _knn_body(posT_ref, posP_ref, idx_ref, *, base):
    a = posT_ref[0]
    pm = posP_ref[0]
    n = pm.shape[1]
    sq_col = jnp.sum(pm * pm, axis=0, keepdims=True)
    sq_row = jnp.sum(a * a, axis=1, keepdims=True)
    d = jax.lax.dot_general(
        a.astype(jnp.bfloat16), pm.astype(jnp.bfloat16),
        (((1,), (0,)), ((), ())),
        preferred_element_type=jnp.float32)
    w = (sq_row + sq_col) - 2.0 * d
    iota = jax.lax.broadcasted_iota(jnp.int32, w.shape, 1)
    for j in range(_NSAMPLE_USED):
        amin = jnp.argmin(w, axis=1).astype(jnp.int32)
        idx_ref[0, :, j] = amin + base
        w = jnp.where(iota == amin[:, None], jnp.float32(jnp.inf), w)


def _knn_pallas(pos, batch_offset=0, tile_rows=256):
    B, C, N = pos.shape
    posP = jnp.pad(pos, ((0, 0), (0, 8 - C), (0, 0)))
    posT8 = posP.transpose(0, 2, 1)
    R = tile_rows
    return pl.pallas_call(
        functools.partial(_knn_body, base=batch_offset),
        grid=(B, N // R),
        in_specs=[
            pl.BlockSpec((1, R, 8), lambda b, t: (b, t, 0)),
            pl.BlockSpec((1, 8, N), lambda b, t: (b, 0, 0)),
        ],
        out_specs=pl.BlockSpec((1, R, _NSAMPLE_USED), lambda b, t: (b, t, 0)),
        out_shape=jax.ShapeDtypeStruct((B, N, _NSAMPLE_USED), jnp.int32),
        compiler_params=pltpu.CompilerParams(
            dimension_semantics=("parallel", "parallel")),
    )(posT8, posP)


def _sc_gather(table, flat_idx):
    num = flat_idx.shape[1]
    window = 128
    mesh = plsc.VectorSubcoreMesh(core_axis_name="core",
                                  subcore_axis_name="subcore")

    @pl.kernel(out_type=jax.ShapeDtypeStruct((num, _GATHER_W), jnp.float32),
               mesh=mesh)
    def gk(x_hbm, i_hbm, o_hbm):
        def body(i_vmem, o_vmem):
            pltpu.sync_copy(x_hbm.at[i_vmem.at[0]], o_vmem)

        pltpu.emit_pipeline(
            body,
            grid=(num // window,),
            in_specs=[pl.BlockSpec((1, window), lambda i: (0, i))],
            out_specs=[pl.BlockSpec((window, _GATHER_W), lambda i: (i, 0))],
            core_axis_name=("core", "subcore"),
            dimension_semantics=(pltpu.PARALLEL,),
        )(i_hbm, o_hbm)

    return gk(table, flat_idx)


def _fold_weights(W1, b1, W2, b2):
    inds_pos = jnp.array([0, 1, 2, 3, 2, 3, 4, 5, 4, 5, 6, 7, 6, 7, 8, 9],
                         dtype=jnp.int32)
    W2m = W2[:, :, 0, 0]

    def lin(gvec):
        g16 = gvec[:, inds_pos].reshape(1, 3, 2, 2, 2, 2)
        tl = g16[:, :, :, 0, :, 0]
        br = g16[:, :, :, 1, :, 1]
        tr = g16[:, :, :, 0, :, 1]
        bl = g16[:, :, :, 1, :, 0]
        xs = jnp.concatenate([tl, br, tr, bl], axis=1)
        h = jnp.einsum('bchw,ochw->bo', xs, W1)[0]
        return W2m @ h

    basis = jnp.eye(30, dtype=jnp.float32).reshape(30, 3, 10)
    cols = jax.vmap(lin)(basis)
    cvec = W2m @ b1 + b2
    rows = jnp.array([(e % 10) * _GATHER_W + e // 10 for e in range(30)],
                     dtype=jnp.int32)
    A = jnp.zeros((_NSAMPLE_USED * _GATHER_W, 64), jnp.float32).at[rows].set(cols)
    return A.T, cvec


def _mlp_body(q_ref, At_ref, c_ref, o_ref):
    acc = c_ref[...]
    r = None
    for j in range(_NSAMPLE_USED):
        pj = jax.lax.dot_general(
            At_ref[:, j * _GATHER_W:(j + 1) * _GATHER_W], q_ref[j],
            (((1,), (1,)), ((), ())),
            preferred_element_type=jnp.float32,
            precision=jax.lax.Precision.HIGHEST)
        r = pj if r is None else r + pj
    o_ref[0] = r + acc


def _mlp_pallas(q3, At, cvec, B, N):
    R2 = 512
    nt = N // R2
    return pl.pallas_call(
        _mlp_body,
        grid=(B, nt),
        in_specs=[
            pl.BlockSpec((_NSAMPLE_USED, R2, _GATHER_W),
                         lambda b, t, nt=nt: (0, b * nt + t, 0)),
            pl.BlockSpec((64, _NSAMPLE_USED * _GATHER_W), lambda b, t: (0, 0)),
            pl.BlockSpec((64, 1), lambda b, t: (0, 0)),
        ],
        out_specs=pl.BlockSpec((1, 64, R2), lambda b, t: (b, 0, t)),
        out_shape=jax.ShapeDtypeStruct((B, 64, N), jnp.float32),
        compiler_params=pltpu.CompilerParams(
            dimension_semantics=("parallel", "parallel")),
    )(q3, At, cvec[:, None])


def kernel(pos, W1, b1, W2, b2):
    B, C, N = pos.shape
    table = jnp.pad(pos.transpose(0, 2, 1).reshape(B * N, C),
                    ((0, 0), (0, _GATHER_W - C)))
    At, cvec = _fold_weights(W1, b1, W2, b2)
    outs = []
    for b in range(B):
        gidx = _knn_pallas(pos[b:b + 1], batch_offset=b * N)
        flat_idx = gidx.reshape(N, _NSAMPLE_USED).T.reshape(1, -1)
        qg = _sc_gather(table, flat_idx)
        q3 = qg.reshape(_NSAMPLE_USED, N, _GATHER_W)
        outs.append(_mlp_pallas(q3, At, cvec, 1, N))
    return jnp.concatenate(outs, axis=0)
